# baseline (speedup 1.0000x reference)
import functools

import jax
import jax.numpy as jnp
from jax import lax
from jax.experimental import pallas as pl
from jax.experimental.pallas import tpu as pltpu
from jax.experimental.pallas import tpu_sc as plsc

B = 1024
J = 26
C = 1000
ROW = J * C
NPTS = B * J
NW = 32
NWORDS = B * ROW
WPW = NWORDS // NW
ZCHUNK = 41600
NZ = WPW // ZCHUNK
PPW = NPTS // NW
SCAT_B = 7

_mesh = plsc.VectorSubcoreMesh(core_axis_name="c", subcore_axis_name="s")


@functools.partial(
    pl.kernel,
    mesh=_mesh,
    out_type=jax.ShapeDtypeStruct((NWORDS,), jnp.int32),
    scratch_types=[
        pltpu.VMEM((ZCHUNK,), jnp.int32),
        pltpu.SemaphoreType.DMA,
    ],
    compiler_params=pltpu.CompilerParams(needs_layout_passes=False),
)
def _zero_fill(zrow_hbm, out_hbm, zbuf, sem):
    wid = lax.axis_index("s") * 2 + lax.axis_index("c")
    base = wid * WPW
    pltpu.sync_copy(zrow_hbm, zbuf)
    handles = []
    for i in range(NZ):
        handles.append(pltpu.async_copy(
            zbuf, out_hbm.at[pl.ds(base + i * ZCHUNK, ZCHUNK)], sem))
    for h in handles:
        h.wait()


@functools.partial(
    pl.kernel,
    mesh=_mesh,
    out_type=(),
    scratch_types=[
        pltpu.VMEM((SCAT_B, 128), jnp.int32),
        pltpu.VMEM((128,), jnp.int32),
        pltpu.SemaphoreType.DMA,
    ],
    compiler_params=pltpu.CompilerParams(needs_layout_passes=False),
)
def _scatter_ones(keys_hbm, out_ref, idxv, ones_v, sem):
    wid = lax.axis_index("s") * 2 + lax.axis_index("c")
    pltpu.sync_copy(keys_hbm.at[wid], idxv)
    ones = jnp.full((16,), 1, jnp.int32)
    for i in range(8):
        ones_v[pl.ds(i * 16, 16)] = ones
    handles = []
    for j in range(SCAT_B):
        handles.append(pltpu.async_copy(
            ones_v, out_ref.at[idxv.at[j]], sem))
    for h in handles:
        h.wait()


def kernel(x):
    xi = x.astype(jnp.int32)
    cpos = xi + (jnp.arange(J, dtype=jnp.int32) * C)[None, :]
    keys = (cpos * B + jnp.arange(B, dtype=jnp.int32)[:, None]).reshape(-1)
    k2 = keys.reshape(NW, PPW)
    pad = jnp.tile(k2[:, -1:], (1, SCAT_B * 128 - PPW))
    k2 = jnp.concatenate([k2, pad], axis=1).reshape(NW, SCAT_B, 128)
    zrow = jnp.zeros((ZCHUNK,), jnp.int32)
    z = _zero_fill(zrow)
    zref = jax.new_ref(z)
    _scatter_ones(k2, zref)
    return zref[...].reshape(ROW, B).T


# speedup vs baseline: 2.3646x; 2.3646x over previous
import functools

import jax
import jax.numpy as jnp
from jax import lax
from jax.experimental import pallas as pl
from jax.experimental.pallas import tpu as pltpu
from jax.experimental.pallas import tpu_sc as plsc

B = 1024
J = 26
C = 1000
ROW = J * C
NPTS = B * J
NW = 32
NWIN = ROW // 8
WPC = 4
NCHUNK = 26
WINS_W = WPC * NCHUNK
CHUNK_R = WPC * 8
KPAD = 16
SENTINEL = 1 << 30

_mesh = plsc.VectorSubcoreMesh(core_axis_name="c", subcore_axis_name="s")


def _worker_starts():
    return jnp.minimum(jnp.arange(NW, dtype=jnp.int32) * NWIN // NW,
                       NWIN - WINS_W)


@functools.partial(
    pl.kernel,
    mesh=_mesh,
    out_type=jax.ShapeDtypeStruct((ROW, B), jnp.int32),
    scratch_types=[
        pltpu.VMEM((NPTS + KPAD,), jnp.int32),
        pltpu.VMEM((NW,), jnp.int32),
        pltpu.VMEM((CHUNK_R, B), jnp.int32),
        pltpu.VMEM((CHUNK_R, B), jnp.int32),
        pltpu.VMEM((CHUNK_R, B), jnp.int32),
        pltpu.SemaphoreType.DMA,
        pltpu.SemaphoreType.DMA,
        pltpu.SemaphoreType.DMA,
    ],
    compiler_params=pltpu.CompilerParams(needs_layout_passes=False),
)
def _onehot_sc(keys_hbm, starts_hbm, zeros_hbm, out_hbm,
               keysv, startsv, buf0, buf1, buf2, s0, s1, s2):
    bufs = (buf0, buf1, buf2)
    sems = (s0, s1, s2)
    wid = lax.axis_index("s") * 2 + lax.axis_index("c")
    s_w = jnp.minimum(wid * NWIN // NW, NWIN - WINS_W)

    pltpu.sync_copy(keys_hbm, keysv.at[pl.ds(0, NPTS)])
    keysv[pl.ds(NPTS, KPAD)] = jnp.full((KPAD,), jnp.int32(SENTINEL))
    pltpu.sync_copy(starts_hbm, startsv)
    pltpu.sync_copy(zeros_hbm, buf0)
    pltpu.sync_copy(zeros_hbm, buf1)
    pltpu.sync_copy(zeros_hbm, buf2)

    ones = jnp.full((16,), 1, jnp.int32)
    zeros_v = jnp.zeros((16,), jnp.int32)

    half = startsv[pl.ds((wid >> 4) * 16, 16)]
    lane = lax.broadcasted_iota(jnp.int32, (16,), 0)
    start_pt = jnp.sum(jnp.where(lane == (wid & 15), half, 0))
    v0 = start_pt >> 4

    def walk_chunk(buf, vstart, k, val):
        c0 = (s_w + WPC * k) * 8
        p0 = c0 * B
        p1 = p0 + CHUNK_R * B

        def cond(v):
            kv = keysv[pl.ds(v * 16, 16)]
            return kv[0] < p1

        def body(v):
            kv = keysv[pl.ds(v * 16, 16)]
            m = (kv >= p0) & (kv < p1)
            lr = (kv >> 10) - c0
            lc = kv & 1023
            plsc.store_scatter(buf, [lr, lc], val, mask=m)
            return v + 1

        vend = lax.while_loop(cond, body, vstart)
        return jnp.maximum(vend - 1, vstart)

    handles = [None, None, None]
    saved = [None, None, None]
    vptr = v0
    for k in range(NCHUNK):
        slot = k % 3
        if handles[slot] is not None:
            handles[slot].wait()
            walk_chunk(bufs[slot], saved[slot], k - 3, zeros_v)
        saved[slot] = vptr
        vptr = walk_chunk(bufs[slot], vptr, k, ones)
        dst = out_hbm.at[pl.ds((s_w + WPC * k) * 8, CHUNK_R)]
        handles[slot] = pltpu.async_copy(bufs[slot], dst, sems[slot])
    handles[0].wait()
    handles[1].wait()
    handles[2].wait()


def kernel(x):
    xi = x.astype(jnp.int32)
    cpos = xi + (jnp.arange(J, dtype=jnp.int32) * C)[None, :]
    keys = cpos * B + jnp.arange(B, dtype=jnp.int32)[:, None]
    keys = jnp.sort(keys.reshape(-1))
    wfirst = _worker_starts() * (8 * B)
    starts = jnp.sum((keys[:, None] < wfirst[None, :]).astype(jnp.int32),
                     axis=0)
    zeros = jnp.zeros((CHUNK_R, B), jnp.int32)
    out_t = _onehot_sc(keys, starts, zeros)
    return out_t.T


# speedup vs baseline: 2.6585x; 1.1243x over previous
import functools

import jax
import jax.numpy as jnp
from jax import lax
from jax.experimental import pallas as pl
from jax.experimental.pallas import tpu as pltpu
from jax.experimental.pallas import tpu_sc as plsc

B = 1024
J = 26
C = 1000
ROW = J * C
NPTS = B * J
NW = 32
NWIN = ROW // 8
WPC = 6
NCHUNK = 17
WINS_W = WPC * NCHUNK
CHUNK_R = WPC * 8
KPAD = 16
SENTINEL = 1 << 30

_mesh = plsc.VectorSubcoreMesh(core_axis_name="c", subcore_axis_name="s")


def _worker_starts():
    return jnp.minimum(jnp.arange(NW, dtype=jnp.int32) * NWIN // NW,
                       NWIN - WINS_W)


@functools.partial(
    pl.kernel,
    mesh=_mesh,
    out_type=jax.ShapeDtypeStruct((ROW, B), jnp.int32),
    scratch_types=[
        pltpu.VMEM((NPTS + KPAD,), jnp.int32),
        pltpu.VMEM((NW,), jnp.int32),
        pltpu.VMEM((CHUNK_R, B), jnp.int32),
        pltpu.VMEM((CHUNK_R, B), jnp.int32),
        pltpu.SemaphoreType.DMA,
        pltpu.SemaphoreType.DMA,
    ],
    compiler_params=pltpu.CompilerParams(needs_layout_passes=False),
)
def _onehot_sc(keys_hbm, starts_hbm, zeros_hbm, out_hbm,
               keysv, startsv, buf0, buf1, s0, s1):
    bufs = (buf0, buf1)
    sems = (s0, s1)
    wid = lax.axis_index("s") * 2 + lax.axis_index("c")
    s_w = jnp.minimum(wid * NWIN // NW, NWIN - WINS_W)

    hk = pltpu.async_copy(keys_hbm, keysv.at[pl.ds(0, NPTS)], s0)
    prime = [pltpu.async_copy(zeros_hbm, buf0, s1),
             pltpu.async_copy(zeros_hbm, buf1, s1)]
    pltpu.sync_copy(starts_hbm, startsv)
    hk.wait()
    keysv[pl.ds(NPTS, KPAD)] = jnp.full((KPAD,), jnp.int32(SENTINEL))

    ones = jnp.full((16,), 1, jnp.int32)
    zeros_v = jnp.zeros((16,), jnp.int32)

    half = startsv[pl.ds((wid >> 4) * 16, 16)]
    lane = lax.broadcasted_iota(jnp.int32, (16,), 0)
    start_pt = jnp.sum(jnp.where(lane == (wid & 15), half, 0))
    v0 = start_pt >> 4

    def walk_chunk(buf, vstart, k, val):
        c0 = (s_w + WPC * k) * 8
        p0 = c0 * B
        p1 = p0 + CHUNK_R * B

        def cond(v):
            kv = keysv[pl.ds(v * 16, 16)]
            return kv[0] < p1

        def body(v):
            kv = keysv[pl.ds(v * 16, 16)]
            m = (kv >= p0) & (kv < p1)
            lr = (kv >> 10) - c0
            lc = kv & 1023
            plsc.store_scatter(buf, [lr, lc], val, mask=m)
            return v + 1

        vend = lax.while_loop(cond, body, vstart)
        return jnp.maximum(vend - 1, vstart)

    handles = [None, None]
    saved = [None, None]
    vptr = v0
    for k in range(NCHUNK):
        slot = k % 2
        if handles[slot] is not None:
            handles[slot].wait()
            walk_chunk(bufs[slot], saved[slot], k - 2, zeros_v)
        else:
            prime[slot].wait()
        saved[slot] = vptr
        vptr = walk_chunk(bufs[slot], vptr, k, ones)
        dst = out_hbm.at[pl.ds((s_w + WPC * k) * 8, CHUNK_R)]
        handles[slot] = pltpu.async_copy(bufs[slot], dst, sems[slot])
    handles[0].wait()
    handles[1].wait()


def kernel(x):
    xi = x.astype(jnp.int32)
    cpos = xi + (jnp.arange(J, dtype=jnp.int32) * C)[None, :]
    keys = cpos * B + jnp.arange(B, dtype=jnp.int32)[:, None]
    keys = jnp.sort(keys.reshape(-1))
    wfirst = _worker_starts() * (8 * B)
    starts = jnp.sum((keys[:, None] < wfirst[None, :]).astype(jnp.int32),
                     axis=0)
    zeros = jnp.zeros((CHUNK_R, B), jnp.int32)
    out_t = _onehot_sc(keys, starts, zeros)
    return out_t.T
